# baseline (speedup 1.0000x reference)
import functools

import jax
import jax.numpy as jnp
import numpy as np
from jax import lax
from jax.experimental import pallas as pl
from jax.experimental.pallas import tpu as pltpu
from jax.experimental.pallas import tpu_sc as plsc

EPS_GIN = 0.1
BN_EPS = 1e-5
LN_EPS = 1e-5

NC = 2
NS = 16
CHUNK = 80


def _sc_agg(src, dst, h, h_edge, n_pad, nchunk):
  hd = h.shape[1]
  ept = nchunk * CHUNK
  rows_per_tile = n_pad // NS
  assert nchunk % 2 == 1

  mesh = plsc.VectorSubcoreMesh(core_axis_name="c", subcore_axis_name="s")

  @functools.partial(
      pl.kernel,
      out_type=jax.ShapeDtypeStruct((NC, n_pad, hd), jnp.float32),
      mesh=mesh,
      scratch_types=[
          pltpu.VMEM((CHUNK,), jnp.int32),
          pltpu.VMEM((CHUNK,), jnp.int32),
          pltpu.VMEM((CHUNK,), jnp.int32),
          pltpu.VMEM((CHUNK,), jnp.int32),
          pltpu.VMEM((CHUNK, hd), jnp.float32),
          pltpu.VMEM((CHUNK, hd), jnp.float32),
          pltpu.VMEM((CHUNK, hd // 2), jnp.int32),
          pltpu.VMEM((CHUNK, hd // 2), jnp.int32),
          pltpu.VMEM((CHUNK,), jnp.int32),
          pltpu.VMEM((CHUNK,), jnp.int32),
          pltpu.VMEM_SHARED((n_pad, hd), jnp.float32),
          pltpu.SemaphoreType.DMA,
          pltpu.SemaphoreType.DMA,
          pltpu.SemaphoreType.DMA,
          pltpu.SemaphoreType.DMA,
          pltpu.SemaphoreType.DMA,
          pltpu.SemaphoreType.DMA,
          pltpu.SemaphoreType.DMA,
          pltpu.SemaphoreType.DMA,
      ],
  )
  def agg_kernel(src_hbm, dst_hbm, h_hbm, he_hbm, out_hbm,
                 sidx0, sidx1, didx0, didx1, hrows0, hrows1, erows0, erows1,
                 dscat0, dscat1, aggsh, isem0, isem1, gsem0, gsem1,
                 esem0, esem1, ssem0, ssem1):
    c = lax.axis_index("c")
    s = lax.axis_index("s")
    wid = c * NS + s
    base_e = wid * ept

    bufs = ((sidx0, didx0, hrows0, erows0, dscat0, isem0, gsem0, esem0,
             ssem0),
            (sidx1, didx1, hrows1, erows1, dscat1, isem1, gsem1, esem1,
             ssem1))

    def issue_idx(k, sidx, didx, isem):
      pltpu.async_copy(src_hbm.at[pl.ds(base_e + k * CHUNK, CHUNK)], sidx,
                       isem)
      pltpu.async_copy(dst_hbm.at[pl.ds(base_e + k * CHUNK, CHUNK)], didx,
                       isem)

    def wait_idx(k, sidx, didx, isem):
      pltpu.make_async_copy(src_hbm.at[pl.ds(base_e + k * CHUNK, CHUNK)],
                            sidx, isem).wait()
      pltpu.make_async_copy(dst_hbm.at[pl.ds(base_e + k * CHUNK, CHUNK)],
                            didx, isem).wait()

    def issue_rows(k, sidx, hrows, erows, gsem, esem):
      pltpu.async_copy(h_hbm.at[sidx], hrows, gsem)
      pltpu.async_copy(he_hbm.at[pl.ds(base_e + k * CHUNK, CHUNK)], erows,
                       esem)

    def wait_rows(k, sidx, hrows, erows, gsem, esem):
      pltpu.make_async_copy(h_hbm.at[sidx], hrows, gsem).wait()
      pltpu.make_async_copy(he_hbm.at[pl.ds(base_e + k * CHUNK, CHUNK)],
                            erows, esem).wait()

    issue_idx(0, sidx0, didx0, isem0)

    def zfill(i, _):
      for j in range(hd // 16):
        hrows0[i, pl.ds(j * 16, 16)] = jnp.zeros((16,), jnp.float32)
      return _
    lax.fori_loop(0, CHUNK, zfill, None)

    def zcopy(k, _):
      pltpu.sync_copy(hrows0, aggsh.at[pl.ds(s * rows_per_tile + k * CHUNK,
                                             CHUNK)])
      return _
    lax.fori_loop(0, rows_per_tile // CHUNK, zcopy, None)

    wait_idx(0, sidx0, didx0, isem0)
    issue_rows(0, sidx0, hrows0, erows0, gsem0, esem0)
    issue_idx(1, sidx1, didx1, isem1)
    plsc.subcore_barrier()

    def step(k, cur, oth):
      sidx, didx, hrows, erows, dscat, isem, gsem, esem, ssem = cur
      (osidx, odidx, ohrows, oerows, odscat, oisem, ogsem, oesem,
       ossem) = oth
      nk = jnp.minimum(k + 1, nchunk - 1)
      nk2 = jnp.minimum(k + 2, nchunk - 1)
      wait_rows(k, sidx, hrows, erows, gsem, esem)
      wait_idx(nk, osidx, odidx, oisem)

      @pl.when(k >= 1)
      def _():
        pltpu.make_async_copy(ohrows, aggsh.at[odscat], ossem).wait()
      issue_rows(nk, osidx, ohrows, oerows, ogsem, oesem)

      for j in range(CHUNK // 16):
        dscat[pl.ds(j * 16, 16)] = didx[pl.ds(j * 16, 16)]

      def row_body(i, _):
        c16 = jnp.full((16,), 16, jnp.int32)
        cmask = jnp.full((16,), -65536, jnp.int32)
        for j in range(hd // 32):
          w = erows[i, pl.ds(j * 16, 16)]
          lo = lax.bitcast_convert_type(jnp.left_shift(w, c16),
                                        jnp.float32)
          hi = lax.bitcast_convert_type(jnp.bitwise_and(w, cmask),
                                        jnp.float32)
          sl0 = pl.ds(j * 32, 16)
          sl1 = pl.ds(j * 32 + 16, 16)
          hrows[i, sl0] = jnp.maximum(hrows[i, sl0] + lo, 0.0)
          hrows[i, sl1] = jnp.maximum(hrows[i, sl1] + hi, 0.0)
        return _
      lax.fori_loop(0, CHUNK, row_body, None)

      pltpu.async_copy(hrows, aggsh.at[dscat], ssem, add=True)
      issue_idx(nk2, sidx, didx, isem)

    def chunk_iter(k, _):
      @pl.when(k % 2 == 0)
      def _():
        step(k, bufs[0], bufs[1])

      @pl.when(k % 2 == 1)
      def _():
        step(k, bufs[1], bufs[0])
      return _
    lax.fori_loop(0, nchunk, chunk_iter, None)

    wait_rows(nchunk - 1, sidx1, hrows1, erows1, gsem1, esem1)
    wait_idx(nchunk - 1, sidx0, didx0, isem0)
    pltpu.make_async_copy(hrows0, aggsh.at[dscat0], ssem0).wait()
    plsc.subcore_barrier()

    pltpu.sync_copy(aggsh.at[pl.ds(s * rows_per_tile, rows_per_tile)],
                    out_hbm.at[c].at[pl.ds(s * rows_per_tile, rows_per_tile)])

  return agg_kernel(src, dst, h, h_edge)


def _encode_kernel(x_ref, w_ref, b_ref, o_ref):
  o_ref[...] = jnp.maximum(
      jnp.dot(x_ref[...], w_ref[...], preferred_element_type=jnp.float32)
      + b_ref[...], 0.0).astype(o_ref.dtype)


def _encode(x, w_t, b, blk, out_dtype=jnp.float32):
  n, _ = x.shape
  h = w_t.shape[1]
  return pl.pallas_call(
      _encode_kernel,
      grid=(n // blk,),
      in_specs=[
          pl.BlockSpec((blk, x.shape[1]), lambda i: (i, 0)),
          pl.BlockSpec(w_t.shape, lambda i: (0, 0)),
          pl.BlockSpec(b.shape, lambda i: (0, 0)),
      ],
      out_specs=pl.BlockSpec((blk, h), lambda i: (i, 0)),
      out_shape=jax.ShapeDtypeStruct((n, h), out_dtype),
  )(x, w_t, b)


def _encode_packed_kernel(x_ref, wl_ref, bl_ref, wh_ref, bh_ref, o_ref):
  x = x_ref[...]
  zl = jnp.maximum(
      jnp.dot(x, wl_ref[...], preferred_element_type=jnp.float32)
      + bl_ref[...], 0.0)
  zh = jnp.maximum(
      jnp.dot(x, wh_ref[...], preferred_element_type=jnp.float32)
      + bh_ref[...], 0.0)
  ul = lax.bitcast_convert_type(zl, jnp.int32)
  uh = lax.bitcast_convert_type(zh, jnp.int32)
  rl = jnp.right_shift(
      ul + 0x7FFF + jnp.bitwise_and(jnp.right_shift(ul, 16), 1), 16)
  rh = jnp.bitwise_and(
      uh + 0x7FFF + jnp.bitwise_and(jnp.right_shift(uh, 16), 1),
      jnp.int32(-65536))
  o_ref[...] = jnp.bitwise_or(rl, rh)


def _encode_packed(x, wl, bl, wh, bh, blk):
  n = x.shape[0]
  hw = wl.shape[1]
  full = lambda a: pl.BlockSpec(a.shape, lambda i: (0, 0))
  return pl.pallas_call(
      _encode_packed_kernel,
      grid=(n // blk,),
      in_specs=[
          pl.BlockSpec((blk, x.shape[1]), lambda i: (i, 0)),
          full(wl), full(bl), full(wh), full(bh),
      ],
      out_specs=pl.BlockSpec((blk, hw), lambda i: (i, 0)),
      out_shape=jax.ShapeDtypeStruct((n, hw), jnp.int32),
  )(x, wl, bl, wh, bh)


def _layer_kernel(h_ref, a0_ref, a1_ref, w1_ref, b1_ref, w2_ref, b2_ref,
                  o_ref):
  z = (1.0 + EPS_GIN) * h_ref[...] + a0_ref[...] + a1_ref[...]
  z = jnp.maximum(
      jnp.dot(z, w1_ref[...], preferred_element_type=jnp.float32)
      + b1_ref[...], 0.0)
  o_ref[...] = jnp.maximum(
      jnp.dot(z, w2_ref[...], preferred_element_type=jnp.float32)
      + b2_ref[...], 0.0)


def _layer_mlp(h, a0, a1, w1_t, b1, w2_t, b2, blk):
  n, hd = h.shape
  full = lambda a: pl.BlockSpec(a.shape, lambda i: (0, 0))
  rows = pl.BlockSpec((blk, hd), lambda i: (i, 0))
  return pl.pallas_call(
      _layer_kernel,
      grid=(n // blk,),
      in_specs=[rows, rows, rows, full(w1_t), full(b1), full(w2_t), full(b2)],
      out_specs=rows,
      out_shape=jax.ShapeDtypeStruct((n, hd), jnp.float32),
  )(h, a0, a1, w1_t, b1, w2_t, b2)


def _final_kernel(h_ref, batch_ref, lng_ref, lnb_ref, ow_ref, ob_ref,
                  p1w_ref, p1b_ref, p2w_ref, p2b_ref,
                  ho_ref, perf_ref, sums_ref, cnts_ref, *, num_groups,
                  nblocks):
  i = pl.program_id(0)
  hv = h_ref[...]
  mu = jnp.mean(hv, axis=1, keepdims=True)
  var = jnp.mean(jnp.square(hv), axis=1, keepdims=True) - jnp.square(mu)
  hn = (hv - mu) * lax.rsqrt(var + LN_EPS) * lng_ref[...] + lnb_ref[...]
  ho = jnp.maximum(
      jnp.dot(hn, ow_ref[...], preferred_element_type=jnp.float32)
      + ob_ref[...], 0.0)
  ho_ref[...] = ho

  bb = batch_ref[0]
  gi = lax.broadcasted_iota(jnp.int32, (num_groups, bb.shape[1]), 0)
  oh = (gi == bb).astype(jnp.float32)

  @pl.when(i == 0)
  def _():
    sums_ref[...] = jnp.zeros_like(sums_ref)
    cnts_ref[...] = jnp.zeros_like(cnts_ref)

  sums_ref[...] += jnp.dot(oh, ho, preferred_element_type=jnp.float32)
  cnts_ref[...] += jnp.sum(oh, axis=1, keepdims=True)

  @pl.when(i == nblocks - 1)
  def _():
    emb = sums_ref[...] / jnp.maximum(cnts_ref[...], 1.0)
    p = jnp.maximum(
        jnp.dot(emb, p1w_ref[...], preferred_element_type=jnp.float32)
        + p1b_ref[...], 0.0)
    perf_ref[...] = (jnp.dot(p, p2w_ref[...],
                             preferred_element_type=jnp.float32)
                     + p2b_ref[...])


def _final(h, batch2d, ln_g, ln_b, ow_t, ob, p1w_t, p1b, p2w_t, p2b, blk):
  n, hd = h.shape
  out_d = ow_t.shape[1]
  num_groups = 64
  nblocks = n // blk
  full = lambda a: pl.BlockSpec(a.shape, lambda i: (0, 0))
  kern = functools.partial(_final_kernel, num_groups=num_groups,
                           nblocks=nblocks)
  return pl.pallas_call(
      kern,
      grid=(nblocks,),
      in_specs=[
          pl.BlockSpec((blk, hd), lambda i: (i, 0)),
          pl.BlockSpec((1, 1, blk), lambda i: (i, 0, 0)),
          full(ln_g), full(ln_b), full(ow_t), full(ob),
          full(p1w_t), full(p1b), full(p2w_t), full(p2b),
      ],
      out_specs=[
          pl.BlockSpec((blk, out_d), lambda i: (i, 0)),
          pl.BlockSpec((num_groups, 1), lambda i: (0, 0)),
      ],
      out_shape=[
          jax.ShapeDtypeStruct((n, out_d), jnp.float32),
          jax.ShapeDtypeStruct((num_groups, 1), jnp.float32),
      ],
      scratch_shapes=[
          pltpu.VMEM((num_groups, out_d), jnp.float32),
          pltpu.VMEM((num_groups, 1), jnp.float32),
      ],
  )(h, batch2d, ln_g, ln_b, ow_t, ob, p1w_t, p1b, p2w_t, p2b)


def kernel(x, edge_index, edge_attr, batch, node_W, node_b, edge_W, edge_b,
           lin1_W, lin1_b, bn_g, bn_b, lin2_W, lin2_b, ln_g, ln_b,
           out_W, out_b, p1_W, p1_b, p2_W, p2_b):
  n, d_node = x.shape
  e = edge_attr.shape[0]
  hd = node_W.shape[0]
  num_layers = lin1_W.shape[0]
  num_groups = 64
  out_d = out_W.shape[0]

  blk = 2048
  n_pad = ((n + blk - 1) // blk) * blk

  x_p = jnp.pad(x, ((0, n_pad - n), (0, 0)))
  batch_p = jnp.pad(batch, (0, n_pad - n), constant_values=num_groups)
  batch2d = batch_p.reshape(n_pad // blk, 1, blk)

  nchunk = e // (NC * NS) // CHUNK
  src = edge_index[0]
  dst = edge_index[1]

  lo_cols = np.concatenate(
      [np.arange(32 * jj, 32 * jj + 16) for jj in range(hd // 32)])
  hi_cols = lo_cols + 16
  h = _encode(x_p, node_W.T, node_b.reshape(1, hd), blk)
  ew_t = edge_W.T
  h_edge = _encode_packed(edge_attr,
                          ew_t[:, lo_cols], edge_b[lo_cols].reshape(1, -1),
                          ew_t[:, hi_cols], edge_b[hi_cols].reshape(1, -1),
                          2560)

  bn_scale = bn_g / jnp.sqrt(1.0 + BN_EPS)
  eff_w1 = jnp.transpose(lin1_W, (0, 2, 1)) * bn_scale[:, None, :]
  eff_b1 = lin1_b * bn_scale + bn_b

  for l in range(num_layers):
    agg = _sc_agg(src, dst, h, h_edge, n_pad, nchunk)
    h = _layer_mlp(h, agg[0], agg[1],
                   eff_w1[l], eff_b1[l].reshape(1, hd),
                   lin2_W[l].T, lin2_b[l].reshape(1, hd), blk)

  h_out, perf = _final(h, batch2d, ln_g.reshape(1, hd), ln_b.reshape(1, hd),
                       out_W.T, out_b.reshape(1, out_d),
                       p1_W.T, p1_b.reshape(1, p1_b.shape[0]),
                       p2_W.T, p2_b.reshape(1, 1), blk)
  return h_out[:n], perf.reshape(num_groups)


# speedup vs baseline: 1.0048x; 1.0048x over previous
import functools

import jax
import jax.numpy as jnp
import numpy as np
from jax import lax
from jax.experimental import pallas as pl
from jax.experimental.pallas import tpu as pltpu
from jax.experimental.pallas import tpu_sc as plsc

EPS_GIN = 0.1
BN_EPS = 1e-5
LN_EPS = 1e-5

NC = 2
NS = 16
CHUNK = 80


def _sc_agg(src, dst, h, h_edge, n_pad, nchunk):
  hd = h.shape[1]
  ept = nchunk * CHUNK
  rows_per_tile = n_pad // NS
  assert nchunk % 2 == 1

  mesh = plsc.VectorSubcoreMesh(core_axis_name="c", subcore_axis_name="s")

  @functools.partial(
      pl.kernel,
      out_type=jax.ShapeDtypeStruct((NC, n_pad, hd), jnp.float32),
      mesh=mesh,
      scratch_types=[
          pltpu.VMEM((CHUNK,), jnp.int32),
          pltpu.VMEM((CHUNK,), jnp.int32),
          pltpu.VMEM((CHUNK,), jnp.int32),
          pltpu.VMEM((CHUNK,), jnp.int32),
          pltpu.VMEM((CHUNK, hd), jnp.float32),
          pltpu.VMEM((CHUNK, hd), jnp.float32),
          pltpu.VMEM((CHUNK, hd), jnp.float32),
          pltpu.VMEM((CHUNK, hd), jnp.float32),
          pltpu.VMEM((CHUNK,), jnp.int32),
          pltpu.VMEM((CHUNK,), jnp.int32),
          pltpu.VMEM_SHARED((n_pad, hd), jnp.float32),
          pltpu.SemaphoreType.DMA,
          pltpu.SemaphoreType.DMA,
          pltpu.SemaphoreType.DMA,
          pltpu.SemaphoreType.DMA,
          pltpu.SemaphoreType.DMA,
          pltpu.SemaphoreType.DMA,
          pltpu.SemaphoreType.DMA,
          pltpu.SemaphoreType.DMA,
      ],
  )
  def agg_kernel(src_hbm, dst_hbm, h_hbm, he_hbm, out_hbm,
                 sidx0, sidx1, didx0, didx1, hrows0, hrows1, erows0, erows1,
                 dscat0, dscat1, aggsh, isem0, isem1, gsem0, gsem1,
                 esem0, esem1, ssem0, ssem1):
    c = lax.axis_index("c")
    s = lax.axis_index("s")
    wid = c * NS + s
    base_e = wid * ept

    bufs = ((sidx0, didx0, hrows0, erows0, dscat0, isem0, gsem0, esem0,
             ssem0),
            (sidx1, didx1, hrows1, erows1, dscat1, isem1, gsem1, esem1,
             ssem1))

    def issue_idx(k, sidx, didx, isem):
      pltpu.async_copy(src_hbm.at[pl.ds(base_e + k * CHUNK, CHUNK)], sidx,
                       isem)
      pltpu.async_copy(dst_hbm.at[pl.ds(base_e + k * CHUNK, CHUNK)], didx,
                       isem)

    def wait_idx(k, sidx, didx, isem):
      pltpu.make_async_copy(src_hbm.at[pl.ds(base_e + k * CHUNK, CHUNK)],
                            sidx, isem).wait()
      pltpu.make_async_copy(dst_hbm.at[pl.ds(base_e + k * CHUNK, CHUNK)],
                            didx, isem).wait()

    def issue_rows(k, sidx, hrows, erows, gsem, esem):
      pltpu.async_copy(h_hbm.at[sidx], hrows, gsem)
      pltpu.async_copy(he_hbm.at[pl.ds(base_e + k * CHUNK, CHUNK)], erows,
                       esem)

    def wait_rows(k, sidx, hrows, erows, gsem, esem):
      pltpu.make_async_copy(h_hbm.at[sidx], hrows, gsem).wait()
      pltpu.make_async_copy(he_hbm.at[pl.ds(base_e + k * CHUNK, CHUNK)],
                            erows, esem).wait()

    issue_idx(0, sidx0, didx0, isem0)

    def zfill(i, _):
      for j in range(hd // 16):
        hrows0[i, pl.ds(j * 16, 16)] = jnp.zeros((16,), jnp.float32)
      return _
    lax.fori_loop(0, CHUNK, zfill, None)

    def zcopy(k, _):
      pltpu.sync_copy(hrows0, aggsh.at[pl.ds(s * rows_per_tile + k * CHUNK,
                                             CHUNK)])
      return _
    lax.fori_loop(0, rows_per_tile // CHUNK, zcopy, None)

    wait_idx(0, sidx0, didx0, isem0)
    issue_rows(0, sidx0, hrows0, erows0, gsem0, esem0)
    issue_idx(1, sidx1, didx1, isem1)
    plsc.subcore_barrier()

    def step(k, cur, oth):
      sidx, didx, hrows, erows, dscat, isem, gsem, esem, ssem = cur
      (osidx, odidx, ohrows, oerows, odscat, oisem, ogsem, oesem,
       ossem) = oth
      nk = jnp.minimum(k + 1, nchunk - 1)
      nk2 = jnp.minimum(k + 2, nchunk - 1)
      wait_rows(k, sidx, hrows, erows, gsem, esem)
      wait_idx(nk, osidx, odidx, oisem)

      @pl.when(k >= 1)
      def _():
        pltpu.make_async_copy(ohrows, aggsh.at[odscat], ossem).wait()
      issue_rows(nk, osidx, ohrows, oerows, ogsem, oesem)

      for j in range(CHUNK // 16):
        dscat[pl.ds(j * 16, 16)] = didx[pl.ds(j * 16, 16)]

      @plsc.parallel_loop(0, CHUNK, step=1, unroll=4)
      def _(i):
        for j in range(hd // 16):
          sl = pl.ds(j * 16, 16)
          hrows[i, sl] = jnp.maximum(hrows[i, sl] + erows[i, sl], 0.0)

      pltpu.async_copy(hrows, aggsh.at[dscat], ssem, add=True)
      issue_idx(nk2, sidx, didx, isem)

    def chunk_iter(k, _):
      @pl.when(k % 2 == 0)
      def _():
        step(k, bufs[0], bufs[1])

      @pl.when(k % 2 == 1)
      def _():
        step(k, bufs[1], bufs[0])
      return _
    lax.fori_loop(0, nchunk, chunk_iter, None)

    wait_rows(nchunk - 1, sidx1, hrows1, erows1, gsem1, esem1)
    wait_idx(nchunk - 1, sidx0, didx0, isem0)
    pltpu.make_async_copy(hrows0, aggsh.at[dscat0], ssem0).wait()
    plsc.subcore_barrier()

    pltpu.sync_copy(aggsh.at[pl.ds(s * rows_per_tile, rows_per_tile)],
                    out_hbm.at[c].at[pl.ds(s * rows_per_tile, rows_per_tile)])

  return agg_kernel(src, dst, h, h_edge)


def _encode_kernel(x_ref, w_ref, b_ref, o_ref):
  o_ref[...] = jnp.maximum(
      jnp.dot(x_ref[...], w_ref[...], preferred_element_type=jnp.float32)
      + b_ref[...], 0.0).astype(o_ref.dtype)


def _encode(x, w_t, b, blk, out_dtype=jnp.float32):
  n, _ = x.shape
  h = w_t.shape[1]
  return pl.pallas_call(
      _encode_kernel,
      grid=(n // blk,),
      in_specs=[
          pl.BlockSpec((blk, x.shape[1]), lambda i: (i, 0)),
          pl.BlockSpec(w_t.shape, lambda i: (0, 0)),
          pl.BlockSpec(b.shape, lambda i: (0, 0)),
      ],
      out_specs=pl.BlockSpec((blk, h), lambda i: (i, 0)),
      out_shape=jax.ShapeDtypeStruct((n, h), out_dtype),
  )(x, w_t, b)


def _layer_kernel(h_ref, a0_ref, a1_ref, w1_ref, b1_ref, w2_ref, b2_ref,
                  o_ref):
  z = (1.0 + EPS_GIN) * h_ref[...] + a0_ref[...] + a1_ref[...]
  z = jnp.maximum(
      jnp.dot(z, w1_ref[...], preferred_element_type=jnp.float32)
      + b1_ref[...], 0.0)
  o_ref[...] = jnp.maximum(
      jnp.dot(z, w2_ref[...], preferred_element_type=jnp.float32)
      + b2_ref[...], 0.0)


def _layer_mlp(h, a0, a1, w1_t, b1, w2_t, b2, blk):
  n, hd = h.shape
  full = lambda a: pl.BlockSpec(a.shape, lambda i: (0, 0))
  rows = pl.BlockSpec((blk, hd), lambda i: (i, 0))
  return pl.pallas_call(
      _layer_kernel,
      grid=(n // blk,),
      in_specs=[rows, rows, rows, full(w1_t), full(b1), full(w2_t), full(b2)],
      out_specs=rows,
      out_shape=jax.ShapeDtypeStruct((n, hd), jnp.float32),
  )(h, a0, a1, w1_t, b1, w2_t, b2)


def _final_kernel(h_ref, batch_ref, lng_ref, lnb_ref, ow_ref, ob_ref,
                  p1w_ref, p1b_ref, p2w_ref, p2b_ref,
                  ho_ref, perf_ref, sums_ref, cnts_ref, *, num_groups,
                  nblocks):
  i = pl.program_id(0)
  hv = h_ref[...]
  mu = jnp.mean(hv, axis=1, keepdims=True)
  var = jnp.mean(jnp.square(hv), axis=1, keepdims=True) - jnp.square(mu)
  hn = (hv - mu) * lax.rsqrt(var + LN_EPS) * lng_ref[...] + lnb_ref[...]
  ho = jnp.maximum(
      jnp.dot(hn, ow_ref[...], preferred_element_type=jnp.float32)
      + ob_ref[...], 0.0)
  ho_ref[...] = ho

  bb = batch_ref[0]
  gi = lax.broadcasted_iota(jnp.int32, (num_groups, bb.shape[1]), 0)
  oh = (gi == bb).astype(jnp.float32)

  @pl.when(i == 0)
  def _():
    sums_ref[...] = jnp.zeros_like(sums_ref)
    cnts_ref[...] = jnp.zeros_like(cnts_ref)

  sums_ref[...] += jnp.dot(oh, ho, preferred_element_type=jnp.float32)
  cnts_ref[...] += jnp.sum(oh, axis=1, keepdims=True)

  @pl.when(i == nblocks - 1)
  def _():
    emb = sums_ref[...] / jnp.maximum(cnts_ref[...], 1.0)
    p = jnp.maximum(
        jnp.dot(emb, p1w_ref[...], preferred_element_type=jnp.float32)
        + p1b_ref[...], 0.0)
    perf_ref[...] = (jnp.dot(p, p2w_ref[...],
                             preferred_element_type=jnp.float32)
                     + p2b_ref[...])


def _final(h, batch2d, ln_g, ln_b, ow_t, ob, p1w_t, p1b, p2w_t, p2b, blk):
  n, hd = h.shape
  out_d = ow_t.shape[1]
  num_groups = 64
  nblocks = n // blk
  full = lambda a: pl.BlockSpec(a.shape, lambda i: (0, 0))
  kern = functools.partial(_final_kernel, num_groups=num_groups,
                           nblocks=nblocks)
  return pl.pallas_call(
      kern,
      grid=(nblocks,),
      in_specs=[
          pl.BlockSpec((blk, hd), lambda i: (i, 0)),
          pl.BlockSpec((1, 1, blk), lambda i: (i, 0, 0)),
          full(ln_g), full(ln_b), full(ow_t), full(ob),
          full(p1w_t), full(p1b), full(p2w_t), full(p2b),
      ],
      out_specs=[
          pl.BlockSpec((blk, out_d), lambda i: (i, 0)),
          pl.BlockSpec((num_groups, 1), lambda i: (0, 0)),
      ],
      out_shape=[
          jax.ShapeDtypeStruct((n, out_d), jnp.float32),
          jax.ShapeDtypeStruct((num_groups, 1), jnp.float32),
      ],
      scratch_shapes=[
          pltpu.VMEM((num_groups, out_d), jnp.float32),
          pltpu.VMEM((num_groups, 1), jnp.float32),
      ],
  )(h, batch2d, ln_g, ln_b, ow_t, ob, p1w_t, p1b, p2w_t, p2b)


def kernel(x, edge_index, edge_attr, batch, node_W, node_b, edge_W, edge_b,
           lin1_W, lin1_b, bn_g, bn_b, lin2_W, lin2_b, ln_g, ln_b,
           out_W, out_b, p1_W, p1_b, p2_W, p2_b):
  n, d_node = x.shape
  e = edge_attr.shape[0]
  hd = node_W.shape[0]
  num_layers = lin1_W.shape[0]
  num_groups = 64
  out_d = out_W.shape[0]

  blk = 2048
  n_pad = ((n + blk - 1) // blk) * blk

  x_p = jnp.pad(x, ((0, n_pad - n), (0, 0)))
  batch_p = jnp.pad(batch, (0, n_pad - n), constant_values=num_groups)
  batch2d = batch_p.reshape(n_pad // blk, 1, blk)

  nchunk = e // (NC * NS) // CHUNK
  src = edge_index[0]
  dst = edge_index[1]

  h = _encode(x_p, node_W.T, node_b.reshape(1, hd), blk)
  h_edge = _encode(edge_attr, edge_W.T, edge_b.reshape(1, hd), 2560)

  bn_scale = bn_g / jnp.sqrt(1.0 + BN_EPS)
  eff_w1 = jnp.transpose(lin1_W, (0, 2, 1)) * bn_scale[:, None, :]
  eff_b1 = lin1_b * bn_scale + bn_b

  for l in range(num_layers):
    agg = _sc_agg(src, dst, h, h_edge, n_pad, nchunk)
    h = _layer_mlp(h, agg[0], agg[1],
                   eff_w1[l], eff_b1[l].reshape(1, hd),
                   lin2_W[l].T, lin2_b[l].reshape(1, hd), blk)

  h_out, perf = _final(h, batch2d, ln_g.reshape(1, hd), ln_b.reshape(1, hd),
                       out_W.T, out_b.reshape(1, out_d),
                       p1_W.T, p1_b.reshape(1, p1_b.shape[0]),
                       p2_W.T, p2_b.reshape(1, 1), blk)
  return h_out[:n], perf.reshape(num_groups)


# speedup vs baseline: 1.1858x; 1.1802x over previous
import functools

import jax
import jax.numpy as jnp
import numpy as np
from jax import lax
from jax.experimental import pallas as pl
from jax.experimental.pallas import tpu as pltpu
from jax.experimental.pallas import tpu_sc as plsc

EPS_GIN = 0.1
BN_EPS = 1e-5
LN_EPS = 1e-5

NC = 2
NS = 16
CHUNK = 80


NBUF = 3


def _sc_pass(table, dst, n_pad, nchunk, gather):
  hd = table.shape[1]
  ept = nchunk * CHUNK
  rows_per_tile = n_pad // NS

  mesh = plsc.VectorSubcoreMesh(core_axis_name="c", subcore_axis_name="s")

  scratch = []
  scratch += [pltpu.VMEM((CHUNK,), jnp.int32)] * (NBUF if gather else 0)
  scratch += [pltpu.VMEM((CHUNK,), jnp.int32)] * NBUF
  scratch += [pltpu.VMEM((CHUNK,), jnp.int32)] * NBUF
  scratch += [pltpu.VMEM((CHUNK, hd), jnp.float32)] * NBUF
  scratch += [pltpu.VMEM_SHARED((n_pad, hd), jnp.float32)]
  scratch += [pltpu.SemaphoreType.DMA] * (3 * NBUF + 1)

  @functools.partial(
      pl.kernel,
      out_type=jax.ShapeDtypeStruct((NC, n_pad, hd), jnp.float32),
      mesh=mesh,
      scratch_types=scratch,
  )
  def pass_kernel(*refs):
    (src_hbm, dst_hbm, table_hbm, out_hbm), rest = refs[:4], list(refs[4:])
    if gather:
      sidx = [rest.pop(0) for _ in range(NBUF)]
    didx = [rest.pop(0) for _ in range(NBUF)]
    dscat = [rest.pop(0) for _ in range(NBUF)]
    rows = [rest.pop(0) for _ in range(NBUF)]
    aggsh = rest.pop(0)
    isem = [rest.pop(0) for _ in range(NBUF)]
    gsem = [rest.pop(0) for _ in range(NBUF)]
    ssem = [rest.pop(0) for _ in range(NBUF)]
    zsem = rest.pop(0)

    c = lax.axis_index("c")
    s = lax.axis_index("s")
    wid = c * NS + s
    base_e = wid * ept

    def issue_idx(k, b):
      if gather:
        pltpu.async_copy(src_hbm.at[pl.ds(base_e + k * CHUNK, CHUNK)],
                         sidx[b], isem[b])
      pltpu.async_copy(dst_hbm.at[pl.ds(base_e + k * CHUNK, CHUNK)],
                       didx[b], isem[b])

    def wait_idx(k, b):
      if gather:
        pltpu.make_async_copy(src_hbm.at[pl.ds(base_e + k * CHUNK, CHUNK)],
                              sidx[b], isem[b]).wait()
      pltpu.make_async_copy(dst_hbm.at[pl.ds(base_e + k * CHUNK, CHUNK)],
                            didx[b], isem[b]).wait()

    def rows_src(k, b):
      if gather:
        return table_hbm.at[sidx[b]]
      return table_hbm.at[pl.ds(base_e + k * CHUNK, CHUNK)]

    def issue_rows(k, b):
      pltpu.async_copy(rows_src(k, b), rows[b], gsem[b])

    def wait_rows(k, b):
      pltpu.make_async_copy(rows_src(k, b), rows[b], gsem[b]).wait()

    issue_idx(0, 0)

    def zfill(i, _):
      for j in range(hd // 16):
        rows[1][i, pl.ds(j * 16, 16)] = jnp.zeros((16,), jnp.float32)
      return _
    lax.fori_loop(0, CHUNK, zfill, None)

    nz = rows_per_tile // CHUNK

    def zissue(k, _):
      pltpu.async_copy(
          rows[1], aggsh.at[pl.ds(s * rows_per_tile + k * CHUNK, CHUNK)],
          zsem)
      return _
    lax.fori_loop(0, nz, zissue, None)

    def zwait(k, _):
      pltpu.make_async_copy(
          rows[1], aggsh.at[pl.ds(s * rows_per_tile + k * CHUNK, CHUNK)],
          zsem).wait()
      return _
    lax.fori_loop(0, nz, zwait, None)

    wait_idx(0, 0)
    issue_rows(0, 0)
    issue_idx(1, 1)
    plsc.subcore_barrier()

    def step(k, b, b1, b2):
      nk = jnp.minimum(k + 1, nchunk - 1)
      nk2 = jnp.minimum(k + 2, nchunk - 1)
      wait_idx(nk, b1)

      @pl.when(k >= 2)
      def _():
        pltpu.make_async_copy(rows[b1], aggsh.at[dscat[b1]],
                              ssem[b1]).wait()
      issue_rows(nk, b1)
      wait_rows(k, b)
      for j in range(CHUNK // 16):
        dscat[b][pl.ds(j * 16, 16)] = didx[b][pl.ds(j * 16, 16)]
      pltpu.async_copy(rows[b], aggsh.at[dscat[b]], ssem[b], add=True)
      issue_idx(nk2, b2)

    def chunk_iter(k, _):
      r = lax.rem(k, NBUF)
      for b in range(NBUF):
        @pl.when(r == b)
        def _(b=b):
          step(k, b, (b + 1) % NBUF, (b + 2) % NBUF)
      return _
    lax.fori_loop(0, nchunk, chunk_iter, None)

    gb = nchunk % NBUF
    ib = (nchunk + 1) % NBUF
    wait_rows(nchunk - 1, gb)
    wait_idx(nchunk - 1, ib)
    for b in ((nchunk - 2) % NBUF, (nchunk - 1) % NBUF):
      pltpu.make_async_copy(rows[b], aggsh.at[dscat[b]], ssem[b]).wait()
    plsc.subcore_barrier()

    pltpu.sync_copy(aggsh.at[pl.ds(s * rows_per_tile, rows_per_tile)],
                    out_hbm.at[c].at[pl.ds(s * rows_per_tile, rows_per_tile)])

  return pass_kernel


def _sc_agg(src, dst, table, n_pad, nchunk, gather):
  k = _sc_pass(table, dst, n_pad, nchunk, gather)
  return k(src, dst, table)


def _encode_kernel(x_ref, w_ref, b_ref, o_ref):
  o_ref[...] = jnp.maximum(
      jnp.dot(x_ref[...], w_ref[...], preferred_element_type=jnp.float32)
      + b_ref[...], 0.0).astype(o_ref.dtype)


def _encode(x, w_t, b, blk, out_dtype=jnp.float32):
  n, _ = x.shape
  h = w_t.shape[1]
  return pl.pallas_call(
      _encode_kernel,
      grid=(n // blk,),
      in_specs=[
          pl.BlockSpec((blk, x.shape[1]), lambda i: (i, 0)),
          pl.BlockSpec(w_t.shape, lambda i: (0, 0)),
          pl.BlockSpec(b.shape, lambda i: (0, 0)),
      ],
      out_specs=pl.BlockSpec((blk, h), lambda i: (i, 0)),
      out_shape=jax.ShapeDtypeStruct((n, h), out_dtype),
  )(x, w_t, b)


def _layer_kernel(h_ref, a0_ref, a1_ref, a2_ref, a3_ref, w1_ref, b1_ref,
                  w2_ref, b2_ref, o_ref):
  z = ((1.0 + EPS_GIN) * h_ref[...] + a0_ref[...] + a1_ref[...]
       + a2_ref[...] + a3_ref[...])
  z = jnp.maximum(
      jnp.dot(z, w1_ref[...], preferred_element_type=jnp.float32)
      + b1_ref[...], 0.0)
  o_ref[...] = jnp.maximum(
      jnp.dot(z, w2_ref[...], preferred_element_type=jnp.float32)
      + b2_ref[...], 0.0)


def _layer_mlp(h, a0, a1, a2, a3, w1_t, b1, w2_t, b2, blk):
  n, hd = h.shape
  full = lambda a: pl.BlockSpec(a.shape, lambda i: (0, 0))
  rows = pl.BlockSpec((blk, hd), lambda i: (i, 0))
  return pl.pallas_call(
      _layer_kernel,
      grid=(n // blk,),
      in_specs=[rows, rows, rows, rows, rows,
                full(w1_t), full(b1), full(w2_t), full(b2)],
      out_specs=rows,
      out_shape=jax.ShapeDtypeStruct((n, hd), jnp.float32),
  )(h, a0, a1, a2, a3, w1_t, b1, w2_t, b2)


def _final_kernel(h_ref, batch_ref, lng_ref, lnb_ref, ow_ref, ob_ref,
                  p1w_ref, p1b_ref, p2w_ref, p2b_ref,
                  ho_ref, perf_ref, sums_ref, cnts_ref, *, num_groups,
                  nblocks):
  i = pl.program_id(0)
  hv = h_ref[...]
  mu = jnp.mean(hv, axis=1, keepdims=True)
  var = jnp.mean(jnp.square(hv), axis=1, keepdims=True) - jnp.square(mu)
  hn = (hv - mu) * lax.rsqrt(var + LN_EPS) * lng_ref[...] + lnb_ref[...]
  ho = jnp.maximum(
      jnp.dot(hn, ow_ref[...], preferred_element_type=jnp.float32)
      + ob_ref[...], 0.0)
  ho_ref[...] = ho

  bb = batch_ref[0]
  gi = lax.broadcasted_iota(jnp.int32, (num_groups, bb.shape[1]), 0)
  oh = (gi == bb).astype(jnp.float32)

  @pl.when(i == 0)
  def _():
    sums_ref[...] = jnp.zeros_like(sums_ref)
    cnts_ref[...] = jnp.zeros_like(cnts_ref)

  sums_ref[...] += jnp.dot(oh, ho, preferred_element_type=jnp.float32)
  cnts_ref[...] += jnp.sum(oh, axis=1, keepdims=True)

  @pl.when(i == nblocks - 1)
  def _():
    emb = sums_ref[...] / jnp.maximum(cnts_ref[...], 1.0)
    p = jnp.maximum(
        jnp.dot(emb, p1w_ref[...], preferred_element_type=jnp.float32)
        + p1b_ref[...], 0.0)
    perf_ref[...] = (jnp.dot(p, p2w_ref[...],
                             preferred_element_type=jnp.float32)
                     + p2b_ref[...])


def _final(h, batch2d, ln_g, ln_b, ow_t, ob, p1w_t, p1b, p2w_t, p2b, blk):
  n, hd = h.shape
  out_d = ow_t.shape[1]
  num_groups = 64
  nblocks = n // blk
  full = lambda a: pl.BlockSpec(a.shape, lambda i: (0, 0))
  kern = functools.partial(_final_kernel, num_groups=num_groups,
                           nblocks=nblocks)
  return pl.pallas_call(
      kern,
      grid=(nblocks,),
      in_specs=[
          pl.BlockSpec((blk, hd), lambda i: (i, 0)),
          pl.BlockSpec((1, 1, blk), lambda i: (i, 0, 0)),
          full(ln_g), full(ln_b), full(ow_t), full(ob),
          full(p1w_t), full(p1b), full(p2w_t), full(p2b),
      ],
      out_specs=[
          pl.BlockSpec((blk, out_d), lambda i: (i, 0)),
          pl.BlockSpec((num_groups, 1), lambda i: (0, 0)),
      ],
      out_shape=[
          jax.ShapeDtypeStruct((n, out_d), jnp.float32),
          jax.ShapeDtypeStruct((num_groups, 1), jnp.float32),
      ],
      scratch_shapes=[
          pltpu.VMEM((num_groups, out_d), jnp.float32),
          pltpu.VMEM((num_groups, 1), jnp.float32),
      ],
  )(h, batch2d, ln_g, ln_b, ow_t, ob, p1w_t, p1b, p2w_t, p2b)


def kernel(x, edge_index, edge_attr, batch, node_W, node_b, edge_W, edge_b,
           lin1_W, lin1_b, bn_g, bn_b, lin2_W, lin2_b, ln_g, ln_b,
           out_W, out_b, p1_W, p1_b, p2_W, p2_b):
  n, d_node = x.shape
  e = edge_attr.shape[0]
  hd = node_W.shape[0]
  num_layers = lin1_W.shape[0]
  num_groups = 64
  out_d = out_W.shape[0]

  blk = 2048
  n_pad = ((n + blk - 1) // blk) * blk

  x_p = jnp.pad(x, ((0, n_pad - n), (0, 0)))
  batch_p = jnp.pad(batch, (0, n_pad - n), constant_values=num_groups)
  batch2d = batch_p.reshape(n_pad // blk, 1, blk)

  nchunk = e // (NC * NS) // CHUNK
  src = edge_index[0]
  dst = edge_index[1]

  h = _encode(x_p, node_W.T, node_b.reshape(1, hd), blk)
  h_edge = _encode(edge_attr, edge_W.T, edge_b.reshape(1, hd), 2560)

  bn_scale = bn_g / jnp.sqrt(1.0 + BN_EPS)
  eff_w1 = jnp.transpose(lin1_W, (0, 2, 1)) * bn_scale[:, None, :]
  eff_b1 = lin1_b * bn_scale + bn_b

  eagg = _sc_agg(src, dst, h_edge, n_pad, nchunk, gather=False)

  for l in range(num_layers):
    agg = _sc_agg(src, dst, h, n_pad, nchunk, gather=True)
    h = _layer_mlp(h, agg[0], agg[1], eagg[0], eagg[1],
                   eff_w1[l], eff_b1[l].reshape(1, hd),
                   lin2_W[l].T, lin2_b[l].reshape(1, hd), blk)

  h_out, perf = _final(h, batch2d, ln_g.reshape(1, hd), ln_b.reshape(1, hd),
                       out_W.T, out_b.reshape(1, out_d),
                       p1_W.T, p1_b.reshape(1, p1_b.shape[0]),
                       p2_W.T, p2_b.reshape(1, 1), blk)
  return h_out[:n], perf.reshape(num_groups)
